# Initial kernel scaffold; baseline (speedup 1.0000x reference)
import jax
import jax.numpy as jnp
from jax.experimental import pallas as pl


def kernel(x, edge_index, W1, a_src1, a_dst1, b1, W2, a_src2, a_dst2, b2, W3, a_src3, a_dst3, b3):
    raise NotImplementedError("write your pallas kernel here")


# speedup vs baseline: 1.0077x; 1.0077x over previous
import functools

import jax
import jax.numpy as jnp
from jax.experimental import pallas as pl
from jax.experimental.pallas import tpu as pltpu


def _matmul_body(x_ref, w_ref, o_ref):
    o_ref[...] = jnp.dot(x_ref[...], w_ref[...],
                         preferred_element_type=jnp.float32)


def _pallas_matmul(x, w, block_rows=1000):
    n, f = x.shape
    k = w.shape[1]
    assert n % block_rows == 0
    grid = (n // block_rows,)
    return pl.pallas_call(
        _matmul_body,
        grid=grid,
        in_specs=[
            pl.BlockSpec((block_rows, f), lambda i: (i, 0)),
            pl.BlockSpec((f, k), lambda i: (0, 0)),
        ],
        out_specs=pl.BlockSpec((block_rows, k), lambda i: (i, 0)),
        out_shape=jax.ShapeDtypeStruct((n, k), jnp.float32),
    )(x, w)


def _segment_softmax(scores, seg, num_segments):
    m = jax.ops.segment_max(scores, seg, num_segments=num_segments)
    m = jnp.where(jnp.isfinite(m), m, 0.0)
    e = jnp.exp(scores - m[seg])
    s = jax.ops.segment_sum(e, seg, num_segments=num_segments)
    return e / (s[seg] + 1e-16)


def _gat_layer(x, src, dst, W, a_src, a_dst, b, heads, out_ch, concat):
    N = x.shape[0]
    wk = W.shape[1]
    h = _pallas_matmul(x, W if wk % 128 == 0 else
                       jnp.pad(W, ((0, 0), (0, 128 - wk % 128))))[:, :wk]
    h = h.reshape(N, heads, out_ch)
    alpha_s = jnp.sum(h * a_src[None, :, :], axis=-1)
    alpha_d = jnp.sum(h * a_dst[None, :, :], axis=-1)
    e = jax.nn.leaky_relu(alpha_s[src] + alpha_d[dst], negative_slope=0.2)
    alpha = _segment_softmax(e, dst, N)
    msg = h[src] * alpha[:, :, None]
    out = jax.ops.segment_sum(msg, dst, num_segments=N)
    if concat:
        out = out.reshape(N, heads * out_ch)
    else:
        out = jnp.mean(out, axis=1)
    return out + b


def kernel(x, edge_index, W1, a_src1, a_dst1, b1, W2, a_src2, a_dst2, b2,
           W3, a_src3, a_dst3, b3):
    N = x.shape[0]
    loop = jnp.arange(N, dtype=edge_index.dtype)
    src = jnp.concatenate([edge_index[0], loop])
    dst = jnp.concatenate([edge_index[1], loop])
    h = _gat_layer(x, src, dst, W1, a_src1, a_dst1, b1, 8, 32, True)
    h = jax.nn.elu(h)
    h = _gat_layer(h, src, dst, W2, a_src2, a_dst2, b2, 8, 32, True)
    h = jax.nn.elu(h)
    out = _gat_layer(h, src, dst, W3, a_src3, a_dst3, b3, 1, 6, False)
    return out


# speedup vs baseline: 32.3777x; 32.1315x over previous
import functools

import jax
import jax.numpy as jnp
from jax import lax
from jax.experimental import pallas as pl
from jax.experimental.pallas import tpu as pltpu
from jax.experimental.pallas import tpu_sc as plsc

F32 = jnp.float32
I32 = jnp.int32

N = 10000
NN = 10112
NT = 632
E_PAD = 331776
PT = E_PAD // 16
NCH = PT // 128
HALF = E_PAD // 2
PT3 = HALF // 16
NCH3 = PT3 // 128


def _dense_body(act, x_ref, bp_ref, w_ref, e_ref, h_ref, asd_ref):
    x = x_ref[...]
    if act:
        x = x + bp_ref[...]
        x = jnp.where(x > 0.0, x, jnp.exp(x) - 1.0)
    h = jnp.dot(x, w_ref[...], preferred_element_type=F32)
    h_ref[...] = h
    asd_ref[...] = jnp.dot(h, e_ref[...], preferred_element_type=F32)


def _dense(x, bprev, W, Easd, act):
    nn, f = x.shape
    k = W.shape[1]
    rb = 1264
    return pl.pallas_call(
        functools.partial(_dense_body, act),
        grid=(nn // rb,),
        in_specs=[pl.BlockSpec((rb, f), lambda i: (i, 0)),
                  pl.BlockSpec((1, f), lambda i: (0, 0)),
                  pl.BlockSpec((f, k), lambda i: (0, 0)),
                  pl.BlockSpec((k, 16), lambda i: (0, 0))],
        out_specs=[pl.BlockSpec((rb, k), lambda i: (i, 0)),
                   pl.BlockSpec((rb, 16), lambda i: (i, 0))],
        out_shape=[jax.ShapeDtypeStruct((nn, k), F32),
                   jax.ShapeDtypeStruct((nn, 16), F32)],
    )(x, bprev.reshape(1, f), W, Easd)


def _prep_body(asd_ref, mx_ref):
    mx_ref[...] = jnp.max(asd_ref[...][:, :8], axis=0, keepdims=True)


def _prep(asd):
    return pl.pallas_call(
        _prep_body,
        out_shape=jax.ShapeDtypeStruct((1, 8), F32),
    )(asd)


def _recip_body(s_ref, r_ref):
    r_ref[...] = 1.0 / (s_ref[...] + 1e-16)


def _recip(s2d):
    return pl.pallas_call(
        _recip_body,
        out_shape=jax.ShapeDtypeStruct(s2d.shape, F32),
    )(s2d)


def _recip3_body(s_ref, r_ref):
    r_ref[...] = 1.0 / (s_ref[0:1, :] + s_ref[1:2, :] + 1e-16)


def _recip3(s3):
    return pl.pallas_call(
        _recip3_body,
        out_shape=jax.ShapeDtypeStruct((1, s3.shape[1]), F32),
    )(s3)


def _comb_body(a_ref, b_ref, o_ref):
    o_ref[...] = a_ref[0] + a_ref[1] + b_ref[...]


def _combine(aggs, b8):
    nn = aggs.shape[1]
    return pl.pallas_call(
        _comb_body,
        out_shape=jax.ShapeDtypeStruct((nn, 8), F32),
    )(aggs, b8.reshape(1, 8))


_MESH = plsc.VectorSubcoreMesh(core_axis_name="c", subcore_axis_name="s")
_SC_PARAMS = pltpu.CompilerParams(needs_layout_passes=False,
                                  use_tc_tiling_on_sc=False)


def _pa8_body(src_h, dst_h, z4_h, as_h, ad_h, mx_h,
              p_h, s_h,
              as_v, ad_v, mx_v, srcb, dstb, pbuf, sem, sacc):
    if True:
        c = lax.axis_index("c")
        t = lax.axis_index("s")
        pltpu.sync_copy(as_h.at[c], as_v)
        pltpu.sync_copy(ad_h.at[c], ad_v)
        pltpu.sync_copy(mx_h, mx_v)
        nd = pl.ds(t * NT, NT)
        pltpu.sync_copy(z4_h.at[nd], sacc.at[nd])
        plsc.subcore_barrier()
        iota = lax.iota(I32, 16)
        mxs = [plsc.load_gather(mx_v, [c * 4 + jnp.full((16,), h, I32)])
               for h in range(4)]

        @pl.loop(0, NCH)
        def _chunk(i):
            base = t * PT + i * 128
            pltpu.sync_copy(src_h.at[pl.ds(base, 128)], srcb)
            pltpu.sync_copy(dst_h.at[pl.ds(base, 128)], dstb)

            @pl.loop(0, 8)
            def _grp(g):
                sv = srcb[pl.ds(g * 16, 16)]
                dv = dstb[pl.ds(g * 16, 16)]
                rows = g * 16 + iota
                sv4 = sv * 4
                dv4 = dv * 4
                for h in range(4):
                    a_s = plsc.load_gather(as_v, [sv4 + h])
                    a_d = plsc.load_gather(ad_v, [dv4 + h])
                    e = a_s + a_d
                    e = jnp.where(e >= 0.0, e, 0.2 * e)
                    em = mxs[h] + a_d
                    m = jnp.where(em >= 0.0, em, 0.2 * em)
                    p = jnp.exp(e - m)
                    plsc.store_scatter(pbuf, [rows, jnp.full((16,), h, I32)], p)

            pltpu.sync_copy(pbuf, p_h.at[c, pl.ds(base, 128)])
            pltpu.async_copy(pbuf, sacc.at[dstb], sem, add=True).wait()

        plsc.subcore_barrier()
        pltpu.sync_copy(sacc.at[nd], s_h.at[c, nd])


def _pass_a8(src, dst, z4, as_f, ad_f, mx8):
    return pl.kernel(
        _pa8_body,
        out_type=[jax.ShapeDtypeStruct((2, E_PAD, 4), F32),
                  jax.ShapeDtypeStruct((2, NN, 4), F32)],
        mesh=_MESH,
        compiler_params=_SC_PARAMS,
        scratch_types=[pltpu.VMEM((NN * 4,), F32),
                       pltpu.VMEM((NN * 4,), F32),
                       pltpu.VMEM((8,), F32),
                       pltpu.VMEM((128,), I32),
                       pltpu.VMEM((128,), I32),
                       pltpu.VMEM((128, 4), F32),
                       pltpu.SemaphoreType.DMA,
                       pltpu.VMEM_SHARED((NN, 4), F32)],
    )(src, dst, z4, as_f, ad_f, mx8)


def _pb8_body(src_h, dst_h, p_h, r_h, h2x_h, z128_h,
              out_h,
              srcb, dstb, gidx, ridx, pb, rbuf, rows, sem, sem2, acc):
    if True:
        c = lax.axis_index("c")
        t = lax.axis_index("s")
        nd = pl.ds(t * NT, NT)
        pltpu.sync_copy(z128_h.at[nd], acc.at[nd])
        plsc.subcore_barrier()
        iota = lax.iota(I32, 16)
        iota4 = lax.bitwise_and(iota, 3)

        @pl.loop(0, NCH)
        def _chunk(i):
            base = t * PT + i * 128
            pltpu.sync_copy(src_h.at[pl.ds(base, 128)], srcb)
            pltpu.sync_copy(dst_h.at[pl.ds(base, 128)], dstb)
            pltpu.sync_copy(p_h.at[c, pl.ds(base, 128)], pb)

            @pl.loop(0, 8)
            def _g(g):
                sl = pl.ds(g * 16, 16)
                gidx[sl] = srcb[sl] * 2 + c
                ridx[sl] = dstb[sl] + c * NN

            d1 = pltpu.async_copy(h2x_h.at[gidx], rows, sem)
            d2 = pltpu.async_copy(r_h.at[ridx], rbuf, sem2)
            d1.wait()
            d2.wait()

            @pl.loop(0, 128)
            def _e(j):
                jj = jnp.full((16,), j, I32)
                pr = plsc.load_gather(pb, [jj, iota4])
                rr = plsc.load_gather(rbuf, [jj, iota4])
                al = pr * rr
                for q in range(4):
                    aq = jnp.broadcast_to(al[q], (16,))
                    s0 = pl.ds(q * 32, 16)
                    s1 = pl.ds(q * 32 + 16, 16)
                    rows[j, s0] = rows[j, s0] * aq
                    rows[j, s1] = rows[j, s1] * aq

            pltpu.async_copy(rows, acc.at[dstb], sem, add=True).wait()

        plsc.subcore_barrier()
        pltpu.sync_copy(acc.at[nd], out_h.at[nd, pl.ds(c * 128, 128)])


def _pass_b8(src, dst, p, r2, h2x, z128):
    return pl.kernel(
        _pb8_body,
        out_type=jax.ShapeDtypeStruct((NN, 256), F32),
        mesh=_MESH,
        compiler_params=_SC_PARAMS,
        scratch_types=[pltpu.VMEM((128,), I32),
                       pltpu.VMEM((128,), I32),
                       pltpu.VMEM((128,), I32),
                       pltpu.VMEM((128,), I32),
                       pltpu.VMEM((128, 4), F32),
                       pltpu.VMEM((128, 4), F32),
                       pltpu.VMEM((128, 128), F32),
                       pltpu.SemaphoreType.DMA,
                       pltpu.SemaphoreType.DMA,
                       pltpu.VMEM_SHARED((NN, 128), F32)],
    )(src, dst, p, r2, h2x, z128)


def _pa1_body(src_h, dst_h, z1_h, as_h, ad_h, mx_h,
              p_h, s_h,
              as_v, ad_v, mx_v, srcb, dstb, pbuf, sem, sacc):
    if True:
        c = lax.axis_index("c")
        t = lax.axis_index("s")
        pltpu.sync_copy(as_h, as_v)
        pltpu.sync_copy(ad_h, ad_v)
        pltpu.sync_copy(mx_h, mx_v)
        nd = pl.ds(t * NT, NT)
        pltpu.sync_copy(z1_h.at[nd], sacc.at[nd])
        plsc.subcore_barrier()
        mx0 = plsc.load_gather(mx_v, [jnp.zeros((16,), I32)])

        @pl.loop(0, NCH3)
        def _chunk(i):
            base = c * HALF + t * PT3 + i * 128
            pltpu.sync_copy(src_h.at[pl.ds(base, 128)], srcb)
            pltpu.sync_copy(dst_h.at[pl.ds(base, 128)], dstb)

            @pl.loop(0, 8)
            def _grp(g):
                sl = pl.ds(g * 16, 16)
                sv = srcb[sl]
                dv = dstb[sl]
                a_s = plsc.load_gather(as_v, [sv])
                a_d = plsc.load_gather(ad_v, [dv])
                e = a_s + a_d
                e = jnp.where(e >= 0.0, e, 0.2 * e)
                em = mx0 + a_d
                m = jnp.where(em >= 0.0, em, 0.2 * em)
                pbuf[sl] = jnp.exp(e - m)

            pltpu.sync_copy(pbuf, p_h.at[pl.ds(base, 128)])
            pltpu.async_copy(pbuf, sacc.at[dstb], sem, add=True).wait()

        plsc.subcore_barrier()
        pltpu.sync_copy(sacc.at[nd], s_h.at[c, nd])


def _pass_a1(src, dst, z1, as_t, ad_t, mx8):
    return pl.kernel(
        _pa1_body,
        out_type=[jax.ShapeDtypeStruct((E_PAD,), F32),
                  jax.ShapeDtypeStruct((2, NN), F32)],
        mesh=_MESH,
        compiler_params=_SC_PARAMS,
        scratch_types=[pltpu.VMEM((NN,), F32),
                       pltpu.VMEM((NN,), F32),
                       pltpu.VMEM((8,), F32),
                       pltpu.VMEM((128,), I32),
                       pltpu.VMEM((128,), I32),
                       pltpu.VMEM((128,), F32),
                       pltpu.SemaphoreType.DMA,
                       pltpu.VMEM_SHARED((NN,), F32)],
    )(src, dst, z1, as_t, ad_t, mx8)


def _pb1_body(src_h, dst_h, p_h, r_h, h8_h, z8_h,
              out_h,
              h8_v, srcb, dstb, pb, rbuf, albuf, msg, sem, acc):
    if True:
        c = lax.axis_index("c")
        t = lax.axis_index("s")
        pltpu.sync_copy(h8_h, h8_v)
        nd = pl.ds(t * NT, NT)
        pltpu.sync_copy(z8_h.at[nd], acc.at[nd])
        plsc.subcore_barrier()
        iota = lax.iota(I32, 16)
        lane_hi = lax.shift_right_logical(iota, 3)
        col8 = lax.bitwise_and(iota, 7)

        @pl.loop(0, NCH3)
        def _chunk(i):
            base = c * HALF + t * PT3 + i * 128
            pltpu.sync_copy(src_h.at[pl.ds(base, 128)], srcb)
            pltpu.sync_copy(dst_h.at[pl.ds(base, 128)], dstb)
            pltpu.sync_copy(p_h.at[pl.ds(base, 128)], pb)
            pltpu.async_copy(r_h.at[dstb], rbuf, sem).wait()

            @pl.loop(0, 8)
            def _g(g):
                sl = pl.ds(g * 16, 16)
                albuf[sl] = pb[sl] * rbuf[sl]

            @pl.loop(0, 64)
            def _v(v):
                eidx = v * 2 + lane_hi
                src16 = plsc.load_gather(srcb, [eidx])
                hv = plsc.load_gather(h8_v, [src16, col8])
                sc = plsc.load_gather(albuf, [eidx])
                plsc.store_scatter(msg, [eidx, col8], hv * sc)

            pltpu.async_copy(msg, acc.at[dstb], sem, add=True).wait()

        plsc.subcore_barrier()
        pltpu.sync_copy(acc.at[nd], out_h.at[c, nd])


def _pass_b1(src, dst, p, r1, h8, z8):
    return pl.kernel(
        _pb1_body,
        out_type=jax.ShapeDtypeStruct((2, NN, 8), F32),
        mesh=_MESH,
        compiler_params=_SC_PARAMS,
        scratch_types=[pltpu.VMEM((NN, 8), F32),
                       pltpu.VMEM((128,), I32),
                       pltpu.VMEM((128,), I32),
                       pltpu.VMEM((128,), F32),
                       pltpu.VMEM((128,), F32),
                       pltpu.VMEM((128,), F32),
                       pltpu.VMEM((128, 8), F32),
                       pltpu.SemaphoreType.DMA,
                       pltpu.VMEM_SHARED((NN, 8), F32)],
    )(src, dst, p, r1, h8, z8)


def _easd(a_s, a_d, k):
    hh, cc = a_s.shape
    eye = jnp.eye(hh, dtype=F32)
    es = (eye[:, None, :] * a_s[:, :, None]).reshape(hh * cc, hh)
    ed = (eye[:, None, :] * a_d[:, :, None]).reshape(hh * cc, hh)
    blk = jnp.zeros((k, 16), F32)
    blk = blk.at[:hh * cc, :hh].set(es)
    blk = blk.at[:hh * cc, 8:8 + hh].set(ed)
    return blk


def _split_flat(arr8):
    return arr8.reshape(NN, 2, 4).transpose(1, 0, 2).reshape(2, NN * 4)


def kernel(x, edge_index, W1, a_src1, a_dst1, b1, W2, a_src2, a_dst2, b2,
           W3, a_src3, a_dst3, b3):
    n = x.shape[0]
    loop = jnp.arange(n, dtype=edge_index.dtype)
    npad = E_PAD - (edge_index.shape[1] + n)
    src = jnp.concatenate([edge_index[0], loop,
                           jnp.arange(npad, dtype=I32) % n])
    dst = jnp.concatenate([edge_index[1], loop,
                           jnp.full((npad,), n, I32)])

    z128 = jnp.zeros((NN, 128), F32)
    z8 = z128[:, :8]
    z4 = z128[:, :4]
    z1 = z128[:, 0]

    x_p = jnp.pad(x, ((0, NN - n), (0, 0)))
    zeros_b = jnp.zeros((x.shape[1],), F32)

    h1, asd1 = _dense(x_p, zeros_b, W1, _easd(a_src1, a_dst1, 256), False)
    mx1 = _prep(asd1).reshape(8)
    p1, s1 = _pass_a8(src, dst, z4, _split_flat(asd1[:, :8]),
                      _split_flat(asd1[:, 8:]), mx1)
    r1 = _recip(s1.reshape(632, 128)).reshape(2 * NN, 4)
    agg1 = _pass_b8(src, dst, p1, r1, h1.reshape(2 * NN, 128), z128)

    h2, asd2 = _dense(agg1, b1, W2, _easd(a_src2, a_dst2, 256), True)
    mx2 = _prep(asd2).reshape(8)
    p2, s2 = _pass_a8(src, dst, z4, _split_flat(asd2[:, :8]),
                      _split_flat(asd2[:, 8:]), mx2)
    r2 = _recip(s2.reshape(632, 128)).reshape(2 * NN, 4)
    agg2 = _pass_b8(src, dst, p2, r2, h2.reshape(2 * NN, 128), z128)

    W3p = jnp.pad(W3, ((0, 0), (0, 128 - W3.shape[1])))
    h3, asd3 = _dense(agg2, b2, W3p, _easd(a_src3, a_dst3, 128), True)
    mx3 = _prep(asd3).reshape(8)
    p3, s3 = _pass_a1(src, dst, z1, asd3[:, 0], asd3[:, 8], mx3)
    r3 = _recip3(s3).reshape(NN)
    agg3 = _pass_b1(src, dst, p3, r3, h3[:, :8], z8)
    out = _combine(agg3, jnp.pad(b3, (0, 2)))
    return out[:n, :6]
